# baseline (speedup 1.0000x reference)
import functools

import jax
import jax.numpy as jnp
from jax import lax
from jax.experimental import pallas as pl
from jax.experimental.pallas import tpu as pltpu
from jax.experimental.pallas import tpu_sc as plsc

VOCAB = 100000
BATCH = 4096
HIST = 200
DIM = 32
N = BATCH * HIST
NC, NS = 2, 16
NW = NC * NS
GATHER = 128
NTC = N // GATHER
TC_PER_W = NTC // NW
K = 10
CHUNK = K * GATHER
NCHUNK = TC_PER_W // K

_mesh = plsc.VectorSubcoreMesh(core_axis_name="c", subcore_axis_name="s")


@functools.partial(
    pl.kernel,
    mesh=_mesh,
    compiler_params=pltpu.CompilerParams(
        use_tc_tiling_on_sc=False, needs_layout_passes=False),
    out_type=jax.ShapeDtypeStruct((N, DIM), jnp.float32),
    scratch_types=[
        pltpu.VMEM((K, GATHER), jnp.int32),
        pltpu.VMEM((K, GATHER), jnp.int32),
        pltpu.VMEM((K, GATHER), jnp.int32),
        pltpu.VMEM((K, GATHER), jnp.int32),
        pltpu.VMEM((K, GATHER), jnp.int32),
        pltpu.VMEM((K, GATHER), jnp.int32),
        pltpu.VMEM((CHUNK, DIM), jnp.float32),
        pltpu.VMEM((CHUNK, DIM), jnp.float32),
        pltpu.SemaphoreType.DMA,
        pltpu.SemaphoreType.DMA,
        pltpu.SemaphoreType.DMA,
        pltpu.SemaphoreType.DMA,
        pltpu.SemaphoreType.DMA,
        pltpu.SemaphoreType.DMA,
    ],
)
def _sc_gather(table_hbm, conc_hbm, time_hbm, out_hbm,
               conc0, conc1, time0, time1, idx0, idx1,
               rows0, rows1,
               isem0, isem1, gsem0, gsem1, osem0, osem1):
    wid = lax.axis_index("s") * NC + lax.axis_index("c")
    base_tc = wid * TC_PER_W

    conc = (conc0, conc1)
    time = (time0, time1)
    idx = (idx0, idx1)
    rows = (rows0, rows1)
    isem = (isem0, isem1)
    gsem = (gsem0, gsem1)
    osem = (osem0, osem1)

    def start_in(ci, b):
        r0 = base_tc + ci * K
        pltpu.async_copy(conc_hbm.at[pl.ds(r0, K)], conc[b], isem[b])
        pltpu.async_copy(time_hbm.at[pl.ds(r0, K)], time[b], isem[b])

    def wait_in(b):
        pltpu.make_async_copy(conc_hbm.at[pl.ds(0, K)], conc[b], isem[b]).wait()
        pltpu.make_async_copy(time_hbm.at[pl.ds(0, K)], time[b], isem[b]).wait()

    def compute_idx(b):
        for j in range(K):
            for i in range(GATHER // 16):
                sl = pl.ds(i * 16, 16)
                idx[b][j, sl] = time[b][j, sl] * VOCAB + conc[b][j, sl]

    def fire_gathers(b):
        for j in range(K):
            pltpu.async_copy(
                table_hbm.at[idx[b].at[j]],
                rows[b].at[pl.ds(j * GATHER, GATHER)],
                gsem[b],
            )

    def wait_gathers(b):
        pltpu.make_async_copy(table_hbm.at[pl.ds(0, CHUNK)], rows[b], gsem[b]).wait()

    def start_out(ci, b):
        n0 = (base_tc + ci * K) * GATHER
        pltpu.async_copy(rows[b], out_hbm.at[pl.ds(n0, CHUNK)], osem[b])

    def wait_out(b):
        pltpu.make_async_copy(rows[b], out_hbm.at[pl.ds(0, CHUNK)], osem[b]).wait()

    start_in(0, 0)
    start_in(1, 1)
    wait_in(0)
    compute_idx(0)
    fire_gathers(0)
    start_in(2, 0)
    wait_in(1)
    compute_idx(1)
    fire_gathers(1)
    start_in(3, 1)
    wait_gathers(0)
    start_out(0, 0)

    def step(ci, b, pb, prefetch):
        wait_in(b)
        compute_idx(b)
        wait_out(b)
        fire_gathers(b)
        if prefetch:
            start_in(ci + 2, b)
        wait_gathers(pb)
        start_out(ci - 1, pb)

    def round_body(r, carry):
        ci = 2 * r
        step(ci, 0, 1, True)
        step(ci + 1, 1, 0, True)
        return carry

    lax.fori_loop(1, NCHUNK // 2 - 1, round_body, 0)

    step(NCHUNK - 2, 0, 1, False)
    step(NCHUNK - 1, 1, 0, False)

    wait_gathers(1)
    start_out(NCHUNK - 1, 1)
    wait_out(0)
    wait_out(1)


def kernel(concept_ids, time_periods, table):
    conc = concept_ids.reshape(N // GATHER, GATHER).astype(jnp.int32)
    time = time_periods.reshape(N // GATHER, GATHER).astype(jnp.int32)
    tbl = jnp.where(table == table, table, jnp.float32(0))
    out = _sc_gather(tbl, conc, time)
    out = jnp.where(out == out, out, jnp.float32(0))
    return out.reshape(BATCH, HIST, DIM)


# speedup vs baseline: 1.5912x; 1.5912x over previous
import functools

import jax
import jax.numpy as jnp
from jax import lax
from jax.experimental import pallas as pl
from jax.experimental.pallas import tpu as pltpu
from jax.experimental.pallas import tpu_sc as plsc

VOCAB = 100000
BATCH = 4096
HIST = 200
DIM = 32
N = BATCH * HIST
NC, NS = 2, 16
NW = NC * NS
GATHER = 128
NTC = N // GATHER
TC_PER_W = NTC // NW
K = 10
CHUNK = K * GATHER
NCHUNK = TC_PER_W // K

_mesh = plsc.VectorSubcoreMesh(core_axis_name="c", subcore_axis_name="s")


@functools.partial(
    pl.kernel,
    mesh=_mesh,
    compiler_params=pltpu.CompilerParams(
        use_tc_tiling_on_sc=False, needs_layout_passes=False),
    out_type=jax.ShapeDtypeStruct((N, DIM), jnp.float32),
    scratch_types=[
        pltpu.VMEM((K, GATHER), jnp.int32),
        pltpu.VMEM((K, GATHER), jnp.int32),
        pltpu.VMEM((K, GATHER), jnp.int32),
        pltpu.VMEM((K, GATHER), jnp.int32),
        pltpu.VMEM((K, GATHER), jnp.int32),
        pltpu.VMEM((K, GATHER), jnp.int32),
        pltpu.VMEM((CHUNK, DIM), jnp.float32),
        pltpu.VMEM((CHUNK, DIM), jnp.float32),
        pltpu.SemaphoreType.DMA,
        pltpu.SemaphoreType.DMA,
        pltpu.SemaphoreType.DMA,
        pltpu.SemaphoreType.DMA,
        pltpu.SemaphoreType.DMA,
        pltpu.SemaphoreType.DMA,
    ],
)
def _sc_gather(table_hbm, conc_hbm, time_hbm, out_hbm,
               conc0, conc1, time0, time1, idx0, idx1,
               rows0, rows1,
               isem0, isem1, gsem0, gsem1, osem0, osem1):
    wid = lax.axis_index("s") * NC + lax.axis_index("c")
    base_tc = wid * TC_PER_W

    conc = (conc0, conc1)
    time = (time0, time1)
    idx = (idx0, idx1)
    rows = (rows0, rows1)
    isem = (isem0, isem1)
    gsem = (gsem0, gsem1)
    osem = (osem0, osem1)

    def start_in(ci, b):
        r0 = base_tc + ci * K
        pltpu.async_copy(conc_hbm.at[pl.ds(r0, K)], conc[b], isem[b])
        pltpu.async_copy(time_hbm.at[pl.ds(r0, K)], time[b], isem[b])

    def wait_in(b):
        pltpu.make_async_copy(conc_hbm.at[pl.ds(0, K)], conc[b], isem[b]).wait()
        pltpu.make_async_copy(time_hbm.at[pl.ds(0, K)], time[b], isem[b]).wait()

    def compute_idx(b):
        for j in range(K):
            for i in range(GATHER // 16):
                sl = pl.ds(i * 16, 16)
                idx[b][j, sl] = time[b][j, sl] * VOCAB + conc[b][j, sl]

    def fire_gathers(b):
        for j in range(K):
            pltpu.async_copy(
                table_hbm.at[idx[b].at[j]],
                rows[b].at[pl.ds(j * GATHER, GATHER)],
                gsem[b],
            )

    def wait_gathers(b):
        pltpu.make_async_copy(table_hbm.at[pl.ds(0, CHUNK)], rows[b], gsem[b]).wait()

    def start_out(ci, b):
        n0 = (base_tc + ci * K) * GATHER
        pltpu.async_copy(rows[b], out_hbm.at[pl.ds(n0, CHUNK)], osem[b])

    def wait_out(b):
        pltpu.make_async_copy(rows[b], out_hbm.at[pl.ds(0, CHUNK)], osem[b]).wait()

    start_in(0, 0)
    start_in(1, 1)
    wait_in(0)
    compute_idx(0)
    fire_gathers(0)
    start_in(2, 0)
    wait_in(1)
    compute_idx(1)
    fire_gathers(1)
    start_in(3, 1)
    wait_gathers(0)
    start_out(0, 0)

    def step(ci, b, pb, prefetch):
        wait_in(b)
        compute_idx(b)
        wait_out(b)
        fire_gathers(b)
        if prefetch:
            start_in(ci + 2, b)
        wait_gathers(pb)
        start_out(ci - 1, pb)

    def round_body(r, carry):
        ci = 2 * r
        step(ci, 0, 1, True)
        step(ci + 1, 1, 0, True)
        return carry

    lax.fori_loop(1, NCHUNK // 2 - 1, round_body, 0)

    step(NCHUNK - 2, 0, 1, False)
    step(NCHUNK - 1, 1, 0, False)

    wait_gathers(1)
    start_out(NCHUNK - 1, 1)
    wait_out(0)
    wait_out(1)


def kernel(concept_ids, time_periods, table):
    conc = concept_ids.reshape(N // GATHER, GATHER).astype(jnp.int32)
    time = time_periods.reshape(N // GATHER, GATHER).astype(jnp.int32)
    out = _sc_gather(table, conc, time)
    return out.reshape(BATCH, HIST, DIM)
